# baseline (speedup 1.0000x reference)
import functools

import jax
import jax.numpy as jnp
from jax import lax
from jax.experimental import pallas as pl
from jax.experimental.pallas import tpu as pltpu
from jax.experimental.pallas import tpu_sc as plsc

DIM = 32
BATCH = 16384
N_NEG = 20
ALPHA = 0.5
GAMMA = 0.3

NC = 2
NS = 16
L = 16
NW = NC * NS
BPW = BATCH // NW
RPW = BPW * N_NEG
CHUNK = 128
NCHUNK = RPW // CHUNK
IDXROWS = BPW // CHUNK


def _sc_body(uidx_hbm, pidx_hbm, nidx_hbm, uemb, p2p, n2p,
             dp_out, dn_out, posp_out, posn_out,
             uidx_v, pidx_v, nidx_v, u_v, pp_v, pn_v,
             negp_v, negn_v, sp_v, sn_v, dp_v, dn_v, sem):
    wid = lax.axis_index("s") * NC + lax.axis_index("c")

    pltpu.sync_copy(uidx_hbm.at[wid], uidx_v)
    pltpu.sync_copy(pidx_hbm.at[wid], pidx_v)
    pltpu.sync_copy(nidx_hbm.at[wid], nidx_v)

    cps = []
    for j in range(IDXROWS):
        dst = pl.ds(j * CHUNK, CHUNK)
        cps.append(pltpu.async_copy(uemb.at[uidx_v.at[j]], u_v.at[dst], sem))
        cps.append(pltpu.async_copy(p2p.at[pidx_v.at[j]], pp_v.at[dst], sem))
        cps.append(pltpu.async_copy(n2p.at[pidx_v.at[j]], pn_v.at[dst], sem))
    for cp in cps:
        cp.wait()

    iota = lax.iota(jnp.int32, L)
    zeros = jnp.zeros((L,), jnp.float32)

    def pos_group(g, carry):
        rows = g * L + iota
        accp = zeros
        accn = zeros
        for d in range(DIM):
            col = jnp.full((L,), d, jnp.int32)
            uv = plsc.load_gather(u_v, [rows, col])
            pv = plsc.load_gather(pp_v, [rows, col])
            nv = plsc.load_gather(pn_v, [rows, col])
            accp = accp + uv * pv
            accn = accn + uv * nv
        sp_v[pl.ds(g * L, L)] = accp
        sn_v[pl.ds(g * L, L)] = accn
        return carry

    lax.fori_loop(0, BPW // L, pos_group, 0)

    def neg_chunk(ci, carry):
        cp1 = pltpu.async_copy(p2p.at[nidx_v.at[ci]], negp_v, sem)
        cp2 = pltpu.async_copy(n2p.at[nidx_v.at[ci]], negn_v, sem)
        cp1.wait()
        cp2.wait()

        def neg_group(g, c2):
            lrow = g * L + iota
            rflat = ci * CHUNK + lrow
            bloc = rflat // N_NEG
            accp = zeros
            accn = zeros
            for d in range(DIM):
                col = jnp.full((L,), d, jnp.int32)
                uv = plsc.load_gather(u_v, [bloc, col])
                pv = plsc.load_gather(negp_v, [lrow, col])
                nv = plsc.load_gather(negn_v, [lrow, col])
                accp = accp + uv * pv
                accn = accn + uv * nv
            posp = plsc.load_gather(sp_v, [bloc])
            posn = plsc.load_gather(sn_v, [bloc])
            off = ci * CHUNK + g * L
            dp_v[pl.ds(off, L)] = posp - accp
            dn_v[pl.ds(off, L)] = posn - accn
            return c2

        lax.fori_loop(0, CHUNK // L, neg_group, 0)
        return carry

    lax.fori_loop(0, NCHUNK, neg_chunk, 0)

    pltpu.sync_copy(dp_v, dp_out.at[pl.ds(wid * RPW, RPW)])
    pltpu.sync_copy(dn_v, dn_out.at[pl.ds(wid * RPW, RPW)])
    pltpu.sync_copy(pp_v, posp_out.at[pl.ds(wid * BPW, BPW)])
    pltpu.sync_copy(pn_v, posn_out.at[pl.ds(wid * BPW, BPW)])


@jax.jit
def _sc_call(uidx, pidx, nidx, uemb, p2p, n2p):
    mesh = plsc.VectorSubcoreMesh(
        core_axis_name="c", subcore_axis_name="s", num_cores=NC, num_subcores=NS
    )
    f = pl.kernel(
        _sc_body,
        out_type=[
            jax.ShapeDtypeStruct((BATCH * N_NEG,), jnp.float32),
            jax.ShapeDtypeStruct((BATCH * N_NEG,), jnp.float32),
            jax.ShapeDtypeStruct((BATCH, DIM), jnp.float32),
            jax.ShapeDtypeStruct((BATCH, DIM), jnp.float32),
        ],
        mesh=mesh,
        scratch_types=[
            pltpu.VMEM((IDXROWS, CHUNK), jnp.int32),
            pltpu.VMEM((IDXROWS, CHUNK), jnp.int32),
            pltpu.VMEM((NCHUNK, CHUNK), jnp.int32),
            pltpu.VMEM((BPW, DIM), jnp.float32),
            pltpu.VMEM((BPW, DIM), jnp.float32),
            pltpu.VMEM((BPW, DIM), jnp.float32),
            pltpu.VMEM((CHUNK, DIM), jnp.float32),
            pltpu.VMEM((CHUNK, DIM), jnp.float32),
            pltpu.VMEM((BPW,), jnp.float32),
            pltpu.VMEM((BPW,), jnp.float32),
            pltpu.VMEM((RPW,), jnp.float32),
            pltpu.VMEM((RPW,), jnp.float32),
            pltpu.SemaphoreType.DMA,
        ],
        compiler_params=pltpu.CompilerParams(
            needs_layout_passes=False, use_tc_tiling_on_sc=False
        ),
    )
    return f(uidx, pidx, nidx, uemb, p2p, n2p)


def _tc_body(dp_ref, dn_ref, pp_ref, pn_ref, w_ref, out_ref):
    lp = jnp.mean(jnp.log(jax.nn.sigmoid(dp_ref[...]) + 1e-10))
    ln = jnp.mean(jnp.log(jax.nn.sigmoid(dn_ref[...]) + 1e-10))
    t = lax.dot_general(
        pn_ref[...], w_ref[...], (((1,), (1,)), ((), ())),
        preferred_element_type=jnp.float32,
        precision=lax.Precision.HIGHEST,
    )
    inter = jnp.sum(pp_ref[...] * t, axis=1)
    li = jnp.mean(jnp.log(jax.nn.sigmoid(inter) + 1e-10))
    out_ref[0, 0] = -(lp + ALPHA * ln + GAMMA * li)


@jax.jit
def _tc_call(dp, dn, posp, posn, W):
    return pl.pallas_call(
        _tc_body,
        out_shape=jax.ShapeDtypeStruct((1, 1), jnp.float32),
        out_specs=pl.BlockSpec(memory_space=pltpu.SMEM),
    )(dp, dn, posp, posn, W)


def kernel(users, items_pos, items_neg, user_emb, item_p2p, item_n2p, W):
    uidx = users.astype(jnp.int32).reshape(NW, IDXROWS, CHUNK)
    pidx = items_pos.astype(jnp.int32).reshape(NW, IDXROWS, CHUNK)
    nidx = items_neg.astype(jnp.int32).reshape(NW, NCHUNK, CHUNK)
    dp, dn, posp, posn = _sc_call(uidx, pidx, nidx, user_emb, item_p2p, item_n2p)
    dp = dp.reshape(BATCH * N_NEG // 128, 128)
    dn = dn.reshape(BATCH * N_NEG // 128, 128)
    out = _tc_call(dp, dn, posp, posn, W)
    return out[0, 0]


# speedup vs baseline: 1.0374x; 1.0374x over previous
import functools

import jax
import jax.numpy as jnp
from jax import lax
from jax.experimental import pallas as pl
from jax.experimental.pallas import tpu as pltpu
from jax.experimental.pallas import tpu_sc as plsc

DIM = 32
BATCH = 16384
N_NEG = 20
ALPHA = 0.5
GAMMA = 0.3

NC = 2
NS = 16
L = 16
NW = NC * NS
BPW = BATCH // NW
RPW = BPW * N_NEG
IROW = 128
CHUNK = 256
NCHUNK = RPW // CHUNK
IDXROWS = BPW // IROW
NIROWS = RPW // IROW
CPI = CHUNK // IROW
GPC = CHUNK // L


def _sc_body(uidx_hbm, pidx_hbm, nidx_hbm, uemb, p2p, n2p,
             dp_out, dn_out, posp_out, posn_out,
             uidx_v, pidx_v, nidx_v, u_v, pp_v, pn_v,
             negp_v, negn_v, sp_v, sn_v, dp_v, dn_v,
             sem, sem0, sem1):
    wid = lax.axis_index("s") * NC + lax.axis_index("c")
    sems = (sem0, sem1)

    pltpu.sync_copy(uidx_hbm.at[wid], uidx_v)
    pltpu.sync_copy(pidx_hbm.at[wid], pidx_v)
    pltpu.sync_copy(nidx_hbm.at[wid], nidx_v)

    def chunk_dmas(c, s):
        out = []
        for tab, buf in ((p2p, negp_v), (n2p, negn_v)):
            for j in range(CPI):
                src = tab.at[nidx_v.at[CPI * c + j]]
                dst = buf.at[s].at[pl.ds(j * IROW, IROW)]
                out.append((src, dst, sems[s]))
        return out

    cps = []
    for j in range(IDXROWS):
        dst = pl.ds(j * IROW, IROW)
        cps.append(pltpu.async_copy(uemb.at[uidx_v.at[j]], u_v.at[dst], sem))
        cps.append(pltpu.async_copy(p2p.at[pidx_v.at[j]], pp_v.at[dst], sem))
        cps.append(pltpu.async_copy(n2p.at[pidx_v.at[j]], pn_v.at[dst], sem))
    for s in range(2):
        for src, dst, sm in chunk_dmas(s, s):
            pltpu.async_copy(src, dst, sm)
    for cp in cps:
        cp.wait()

    iota = lax.iota(jnp.int32, L)
    zeros = jnp.zeros((L,), jnp.float32)

    def pos_group(g, carry):
        rows = g * L + iota
        accp = zeros
        accn = zeros
        for d in range(DIM):
            col = jnp.full((L,), d, jnp.int32)
            uv = plsc.load_gather(u_v, [rows, col])
            pv = plsc.load_gather(pp_v, [rows, col])
            nv = plsc.load_gather(pn_v, [rows, col])
            accp = accp + uv * pv
            accn = accn + uv * nv
        sp_v[pl.ds(g * L, L)] = accp
        sn_v[pl.ds(g * L, L)] = accn
        return carry

    lax.fori_loop(0, BPW // L, pos_group, 0)

    def chunk_compute(c, s):
        for src, dst, sm in chunk_dmas(c, s):
            pltpu.make_async_copy(src, dst, sm).wait()

        def neg_group(g, c2):
            lrow = g * L + iota
            rflat = c * CHUNK + lrow
            bloc = rflat // N_NEG
            accp = zeros
            accn = zeros
            for d in range(DIM):
                col = jnp.full((L,), d, jnp.int32)
                uv = plsc.load_gather(u_v, [bloc, col])
                pv = plsc.load_gather(negp_v.at[s], [lrow, col])
                nv = plsc.load_gather(negn_v.at[s], [lrow, col])
                accp = accp + uv * pv
                accn = accn + uv * nv
            posp = plsc.load_gather(sp_v, [bloc])
            posn = plsc.load_gather(sn_v, [bloc])
            off = c * CHUNK + g * L
            dp_v[pl.ds(off, L)] = posp - accp
            dn_v[pl.ds(off, L)] = posn - accn
            return c2

        lax.fori_loop(0, GPC, neg_group, 0)

    def chunk_pair(k, carry):
        for s in range(2):
            c = 2 * k + s
            chunk_compute(c, s)
            cnext = c + 2

            @pl.when(cnext < NCHUNK)
            def _():
                for src, dst, sm in chunk_dmas(cnext, s):
                    pltpu.async_copy(src, dst, sm)

        return carry

    lax.fori_loop(0, NCHUNK // 2, chunk_pair, 0)

    pltpu.sync_copy(dp_v, dp_out.at[pl.ds(wid * RPW, RPW)])
    pltpu.sync_copy(dn_v, dn_out.at[pl.ds(wid * RPW, RPW)])
    pltpu.sync_copy(pp_v, posp_out.at[pl.ds(wid * BPW, BPW)])
    pltpu.sync_copy(pn_v, posn_out.at[pl.ds(wid * BPW, BPW)])


@jax.jit
def _sc_call(uidx, pidx, nidx, uemb, p2p, n2p):
    mesh = plsc.VectorSubcoreMesh(
        core_axis_name="c", subcore_axis_name="s", num_cores=NC, num_subcores=NS
    )
    f = pl.kernel(
        _sc_body,
        out_type=[
            jax.ShapeDtypeStruct((BATCH * N_NEG,), jnp.float32),
            jax.ShapeDtypeStruct((BATCH * N_NEG,), jnp.float32),
            jax.ShapeDtypeStruct((BATCH, DIM), jnp.float32),
            jax.ShapeDtypeStruct((BATCH, DIM), jnp.float32),
        ],
        mesh=mesh,
        scratch_types=[
            pltpu.VMEM((IDXROWS, IROW), jnp.int32),
            pltpu.VMEM((IDXROWS, IROW), jnp.int32),
            pltpu.VMEM((NIROWS, IROW), jnp.int32),
            pltpu.VMEM((BPW, DIM), jnp.float32),
            pltpu.VMEM((BPW, DIM), jnp.float32),
            pltpu.VMEM((BPW, DIM), jnp.float32),
            pltpu.VMEM((2, CHUNK, DIM), jnp.float32),
            pltpu.VMEM((2, CHUNK, DIM), jnp.float32),
            pltpu.VMEM((BPW,), jnp.float32),
            pltpu.VMEM((BPW,), jnp.float32),
            pltpu.VMEM((RPW,), jnp.float32),
            pltpu.VMEM((RPW,), jnp.float32),
            pltpu.SemaphoreType.DMA,
            pltpu.SemaphoreType.DMA,
            pltpu.SemaphoreType.DMA,
        ],
        compiler_params=pltpu.CompilerParams(
            needs_layout_passes=False, use_tc_tiling_on_sc=False
        ),
    )
    return f(uidx, pidx, nidx, uemb, p2p, n2p)


def _tc_body(dp_ref, dn_ref, pp_ref, pn_ref, w_ref, out_ref):
    lp = jnp.mean(jnp.log(jax.nn.sigmoid(dp_ref[...]) + 1e-10))
    ln = jnp.mean(jnp.log(jax.nn.sigmoid(dn_ref[...]) + 1e-10))
    t = lax.dot_general(
        pn_ref[...], w_ref[...], (((1,), (1,)), ((), ())),
        preferred_element_type=jnp.float32,
        precision=lax.Precision.HIGHEST,
    )
    inter = jnp.sum(pp_ref[...] * t, axis=1)
    li = jnp.mean(jnp.log(jax.nn.sigmoid(inter) + 1e-10))
    out_ref[0, 0] = -(lp + ALPHA * ln + GAMMA * li)


@jax.jit
def _tc_call(dp, dn, posp, posn, W):
    return pl.pallas_call(
        _tc_body,
        out_shape=jax.ShapeDtypeStruct((1, 1), jnp.float32),
        out_specs=pl.BlockSpec(memory_space=pltpu.SMEM),
    )(dp, dn, posp, posn, W)


def kernel(users, items_pos, items_neg, user_emb, item_p2p, item_n2p, W):
    uidx = users.astype(jnp.int32).reshape(NW, IDXROWS, IROW)
    pidx = items_pos.astype(jnp.int32).reshape(NW, IDXROWS, IROW)
    nidx = items_neg.astype(jnp.int32).reshape(NW, NIROWS, IROW)
    dp, dn, posp, posn = _sc_call(uidx, pidx, nidx, user_emb, item_p2p, item_n2p)
    dp = dp.reshape(BATCH * N_NEG // 128, 128)
    dn = dn.reshape(BATCH * N_NEG // 128, 128)
    out = _tc_call(dp, dn, posp, posn, W)
    return out[0, 0]


# speedup vs baseline: 1.3335x; 1.2855x over previous
import functools

import jax
import jax.numpy as jnp
from jax import lax
from jax.experimental import pallas as pl
from jax.experimental.pallas import tpu as pltpu
from jax.experimental.pallas import tpu_sc as plsc
from jax.experimental import layout as jax_layout

DIM = 32
BATCH = 16384
N_NEG = 20
ALPHA = 0.5
GAMMA = 0.3

NC = 2
NS = 16
L = 16
NW = NC * NS
BPW = BATCH // NW
RPW = BPW * N_NEG
IROW = 128
CHUNK = 256
NCHUNK = RPW // CHUNK
IDXROWS = BPW // IROW
NIROWS = RPW // IROW
CPI = CHUNK // IROW
GPC = CHUNK // L


def _sc_body(uidx_hbm, pidx_hbm, nidx_hbm, uemb, p2p, n2p,
             dp_out, dn_out, posp_out, posn_out,
             uidx_v, pidx_v, nidx_v, u_v, pp_v, pn_v,
             negp_v, negn_v, sp_v, sn_v, dp_v, dn_v,
             sem, sem0, sem1):
    wid = lax.axis_index("s") * NC + lax.axis_index("c")
    sems = (sem0, sem1)

    pltpu.sync_copy(uidx_hbm.at[wid], uidx_v)
    pltpu.sync_copy(pidx_hbm.at[wid], pidx_v)
    pltpu.sync_copy(nidx_hbm.at[wid], nidx_v)

    def chunk_dmas(c, s):
        out = []
        for tab, buf in ((p2p, negp_v), (n2p, negn_v)):
            for j in range(CPI):
                src = tab.at[nidx_v.at[CPI * c + j]]
                dst = buf.at[s].at[pl.ds(j * IROW, IROW)]
                out.append((src, dst, sems[s]))
        return out

    cps = []
    for j in range(IDXROWS):
        dst = pl.ds(j * IROW, IROW)
        cps.append(pltpu.async_copy(uemb.at[uidx_v.at[j]], u_v.at[dst], sem))
        cps.append(pltpu.async_copy(p2p.at[pidx_v.at[j]], pp_v.at[dst], sem))
        cps.append(pltpu.async_copy(n2p.at[pidx_v.at[j]], pn_v.at[dst], sem))
    for s in range(2):
        for src, dst, sm in chunk_dmas(s, s):
            pltpu.async_copy(src, dst, sm)
    for cp in cps:
        cp.wait()

    iota = lax.iota(jnp.int32, L)
    zeros = jnp.zeros((L,), jnp.float32)

    def pos_group(g, carry):
        rows = g * L + iota
        accp = zeros
        accn = zeros
        for d in range(DIM):
            col = jnp.full((L,), d, jnp.int32)
            uv = plsc.load_gather(u_v, [rows, col])
            pv = plsc.load_gather(pp_v, [rows, col])
            nv = plsc.load_gather(pn_v, [rows, col])
            accp = accp + uv * pv
            accn = accn + uv * nv
        sp_v[pl.ds(g * L, L)] = accp
        sn_v[pl.ds(g * L, L)] = accn
        return carry

    lax.fori_loop(0, BPW // L, pos_group, 0)

    def chunk_compute(c, s):
        for src, dst, sm in chunk_dmas(c, s):
            pltpu.make_async_copy(src, dst, sm).wait()

        def neg_group(g, c2):
            lrow = g * L + iota
            rflat = c * CHUNK + lrow
            bloc = rflat // N_NEG
            accp = zeros
            accn = zeros
            for d in range(DIM):
                col = jnp.full((L,), d, jnp.int32)
                uv = plsc.load_gather(u_v, [bloc, col])
                pv = plsc.load_gather(negp_v.at[s], [lrow, col])
                nv = plsc.load_gather(negn_v.at[s], [lrow, col])
                accp = accp + uv * pv
                accn = accn + uv * nv
            posp = plsc.load_gather(sp_v, [bloc])
            posn = plsc.load_gather(sn_v, [bloc])
            off = c * CHUNK + g * L
            dp_v[pl.ds(off, L)] = posp - accp
            dn_v[pl.ds(off, L)] = posn - accn
            return c2

        lax.fori_loop(0, GPC, neg_group, 0)

    def chunk_pair(k, carry):
        for s in range(2):
            c = 2 * k + s
            chunk_compute(c, s)
            cnext = c + 2

            @pl.when(cnext < NCHUNK)
            def _():
                for src, dst, sm in chunk_dmas(cnext, s):
                    pltpu.async_copy(src, dst, sm)

        return carry

    lax.fori_loop(0, NCHUNK // 2, chunk_pair, 0)

    pltpu.sync_copy(dp_v, dp_out.at[pl.ds(wid * RPW, RPW)])
    pltpu.sync_copy(dn_v, dn_out.at[pl.ds(wid * RPW, RPW)])
    pltpu.sync_copy(pp_v, posp_out.at[pl.ds(wid * BPW, BPW)])
    pltpu.sync_copy(pn_v, posn_out.at[pl.ds(wid * BPW, BPW)])


@jax.jit
def _sc_call(uidx, pidx, nidx, uemb, p2p, n2p):
    mesh = plsc.VectorSubcoreMesh(
        core_axis_name="c", subcore_axis_name="s", num_cores=NC, num_subcores=NS
    )
    f = pl.kernel(
        _sc_body,
        out_type=[
            jax.ShapeDtypeStruct((BATCH * N_NEG,), jnp.float32),
            jax.ShapeDtypeStruct((BATCH * N_NEG,), jnp.float32),
            jax.ShapeDtypeStruct((BATCH, DIM), jnp.float32),
            jax.ShapeDtypeStruct((BATCH, DIM), jnp.float32),
        ],
        mesh=mesh,
        scratch_types=[
            pltpu.VMEM((IDXROWS, IROW), jnp.int32),
            pltpu.VMEM((IDXROWS, IROW), jnp.int32),
            pltpu.VMEM((NIROWS, IROW), jnp.int32),
            pltpu.VMEM((BPW, DIM), jnp.float32),
            pltpu.VMEM((BPW, DIM), jnp.float32),
            pltpu.VMEM((BPW, DIM), jnp.float32),
            pltpu.VMEM((2, CHUNK, DIM), jnp.float32),
            pltpu.VMEM((2, CHUNK, DIM), jnp.float32),
            pltpu.VMEM((BPW,), jnp.float32),
            pltpu.VMEM((BPW,), jnp.float32),
            pltpu.VMEM((RPW,), jnp.float32),
            pltpu.VMEM((RPW,), jnp.float32),
            pltpu.SemaphoreType.DMA,
            pltpu.SemaphoreType.DMA,
            pltpu.SemaphoreType.DMA,
        ],
        compiler_params=pltpu.CompilerParams(
            needs_layout_passes=False, use_tc_tiling_on_sc=False
        ),
    )
    return f(uidx, pidx, nidx, uemb, p2p, n2p)


def _tc_body(dp_ref, dn_ref, pp_ref, pn_ref, w_ref, out_ref):
    lp = jnp.mean(jnp.log(jax.nn.sigmoid(dp_ref[...]) + 1e-10))
    ln = jnp.mean(jnp.log(jax.nn.sigmoid(dn_ref[...]) + 1e-10))
    t = lax.dot_general(
        pn_ref[...], w_ref[...], (((1,), (1,)), ((), ())),
        preferred_element_type=jnp.float32,
        precision=lax.Precision.HIGHEST,
    )
    inter = jnp.sum(pp_ref[...] * t, axis=1)
    li = jnp.mean(jnp.log(jax.nn.sigmoid(inter) + 1e-10))
    out_ref[0, 0] = -(lp + ALPHA * ln + GAMMA * li)


@jax.jit
def _tc_call(dp, dn, posp, posn, W):
    return pl.pallas_call(
        _tc_body,
        out_shape=jax.ShapeDtypeStruct((1, 1), jnp.float32),
        out_specs=pl.BlockSpec(memory_space=pltpu.SMEM),
    )(dp, dn, posp, posn, W)


def kernel(users, items_pos, items_neg, user_emb, item_p2p, item_n2p, W):
    uidx = users.astype(jnp.int32).reshape(NW, IDXROWS, IROW)
    pidx = items_pos.astype(jnp.int32).reshape(NW, IDXROWS, IROW)
    nidx = items_neg.astype(jnp.int32).reshape(NW, NIROWS, IROW)
    row_major = jax_layout.Layout(major_to_minor=(0, 1))
    uemb = jax_layout.with_layout_constraint(user_emb, row_major)
    p2p = jax_layout.with_layout_constraint(item_p2p, row_major)
    n2p = jax_layout.with_layout_constraint(item_n2p, row_major)
    dp, dn, posp, posn = _sc_call(uidx, pidx, nidx, uemb, p2p, n2p)
    dp = dp.reshape(BATCH * N_NEG // 128, 128)
    dn = dn.reshape(BATCH * N_NEG // 128, 128)
    out = _tc_call(dp, dn, posp, posn, W)
    return out[0, 0]
